# Initial kernel scaffold; baseline (speedup 1.0000x reference)
import functools

import jax
import jax.numpy as jnp
from jax import lax
from jax.experimental import pallas as pl
from jax.experimental.pallas import tpu as pltpu
from jax.experimental.pallas import tpu_sc as plsc

_IDXW = 125
_GROUP = 4
_NSUB = 16
_NCORE = 2

_MESH = plsc.VectorSubcoreMesh(
    core_axis_name="c", subcore_axis_name="s",
    num_cores=_NCORE, num_subcores=_NSUB)


def _node_precompute(static_nodes, dynamic_nodes, W1, F0):
    N, S = static_nodes.shape
    D = dynamic_nodes.shape[1]
    H = W1.shape[1]
    BN = 1000

    def body(s_ref, d_ref, w1_ref, f0_ref, trow_ref, tcol_ref, h0_ref):
        s = s_ref[...]
        d = d_ref[...]
        w1 = w1_ref[...]
        f32 = jnp.float32
        prow = (jnp.dot(s, w1[0:S], preferred_element_type=f32)
                + jnp.dot(d, w1[2 * S:2 * S + D], preferred_element_type=f32))
        pcol = (jnp.dot(s, w1[S:2 * S], preferred_element_type=f32)
                + jnp.dot(d, w1[2 * S + D:2 * S + 2 * D], preferred_element_type=f32))
        h0 = jnp.dot(d, f0_ref[...], preferred_element_type=f32)
        trow_ref[:, 0:H] = prow
        trow_ref[:, H:H + D] = h0
        tcol_ref[:, 0:H] = pcol
        tcol_ref[:, H:H + D] = h0
        h0_ref[...] = h0

    return pl.pallas_call(
        body,
        grid=(N // BN,),
        in_specs=[
            pl.BlockSpec((BN, S), lambda i: (i, 0)),
            pl.BlockSpec((BN, D), lambda i: (i, 0)),
            pl.BlockSpec(W1.shape, lambda i: (0, 0)),
            pl.BlockSpec((D, D), lambda i: (0, 0)),
        ],
        out_specs=[
            pl.BlockSpec((BN, H + D), lambda i: (i, 0)),
            pl.BlockSpec((BN, H + D), lambda i: (i, 0)),
            pl.BlockSpec((BN, D), lambda i: (i, 0)),
        ],
        out_shape=[
            jax.ShapeDtypeStruct((N, H + D), jnp.float32),
            jax.ShapeDtypeStruct((N, H + D), jnp.float32),
            jax.ShapeDtypeStruct((N, D), jnp.float32),
        ],
    )(static_nodes, dynamic_nodes, W1, F0)


def _edge_mlp_flux(gr, gc, edge_features, w1e, b1, W2, b2):
    E = gr.shape[0]
    H = W2.shape[0]
    D = W2.shape[1]
    EF = edge_features.shape[1]
    BE = 8000

    def body(gr_ref, gc_ref, ef_ref, w1e_ref, b1_ref, w2_ref, b2_ref,
             msg_ref, flux_ref):
        grb = gr_ref[...]
        gcb = gc_ref[...]
        efb = ef_ref[...]
        pre = grb[:, 0:H] + gcb[:, 0:H] + b1_ref[...]
        for i in range(EF):
            pre = pre + efb[:, i:i + 1] * w1e_ref[i:i + 1, :]
        m = jnp.dot(jnp.maximum(pre, 0.0), w2_ref[...],
                    preferred_element_type=jnp.float32) + b2_ref[...]
        nrm = jnp.sqrt(jnp.sum(m * m, axis=1, keepdims=True))
        msg = m / nrm
        msg = jnp.where(jnp.isnan(msg), 0.0, msg)
        hr = grb[:, H:H + D]
        hc = gcb[:, H:H + D]
        mask = ((jnp.sum(hr, axis=1) != 0.0)
                | (jnp.sum(hc, axis=1) != 0.0))
        flux = jnp.where(mask[:, None], (hc - hr) * msg, 0.0)
        msg_ref[...] = msg
        flux_ref[...] = flux

    return pl.pallas_call(
        body,
        grid=(E // BE,),
        in_specs=[
            pl.BlockSpec((BE, H + D), lambda i: (i, 0)),
            pl.BlockSpec((BE, H + D), lambda i: (i, 0)),
            pl.BlockSpec((BE, EF), lambda i: (i, 0)),
            pl.BlockSpec((EF, H), lambda i: (0, 0)),
            pl.BlockSpec((1, H), lambda i: (0, 0)),
            pl.BlockSpec((H, D), lambda i: (0, 0)),
            pl.BlockSpec((1, D), lambda i: (0, 0)),
        ],
        out_specs=[
            pl.BlockSpec((BE, D), lambda i: (i, 0)),
            pl.BlockSpec((BE, D), lambda i: (i, 0)),
        ],
        out_shape=[
            jax.ShapeDtypeStruct((E, D), jnp.float32),
            jax.ShapeDtypeStruct((E, D), jnp.float32),
        ],
    )(gr, gc, edge_features, w1e, b1, W2, b2)


def _flux_only(hr, hc, msg):
    E, D = hr.shape
    BE = 8000

    def body(hr_ref, hc_ref, msg_ref, flux_ref):
        hrb = hr_ref[...]
        hcb = hc_ref[...]
        mask = ((jnp.sum(hrb, axis=1) != 0.0)
                | (jnp.sum(hcb, axis=1) != 0.0))
        flux_ref[...] = jnp.where(mask[:, None], (hcb - hrb) * msg_ref[...], 0.0)

    return pl.pallas_call(
        body,
        grid=(E // BE,),
        in_specs=[pl.BlockSpec((BE, D), lambda i: (i, 0))] * 3,
        out_specs=pl.BlockSpec((BE, D), lambda i: (i, 0)),
        out_shape=jax.ShapeDtypeStruct((E, D), jnp.float32),
    )(hr, hc, msg)


def _update_h(h, p0, p1, F):
    N, D = h.shape
    BN = 1000

    def body(h_ref, p0_ref, p1_ref, f_ref, out_ref):
        agg = p0_ref[...] + p1_ref[...]
        out_ref[...] = h_ref[...] + jnp.dot(
            agg, f_ref[...], preferred_element_type=jnp.float32)

    return pl.pallas_call(
        body,
        grid=(N // BN,),
        in_specs=[
            pl.BlockSpec((BN, D), lambda i: (i, 0)),
            pl.BlockSpec((BN, D), lambda i: (i, 0)),
            pl.BlockSpec((BN, D), lambda i: (i, 0)),
            pl.BlockSpec((D, D), lambda i: (0, 0)),
        ],
        out_specs=pl.BlockSpec((BN, D), lambda i: (i, 0)),
        out_shape=jax.ShapeDtypeStruct((N, D), jnp.float32),
    )(h, p0, p1, F)


def _sc_gather(table_row, table_col, row2d, col2d):
    D = table_row.shape[1]
    E = row2d.shape[0] * row2d.shape[1]
    EW = E // _NSUB
    RW = EW // _IDXW
    NG = RW // _GROUP
    GE = _GROUP * _IDXW

    @functools.partial(
        pl.kernel, mesh=_MESH,
        out_type=[jax.ShapeDtypeStruct((E, D), jnp.float32),
                  jax.ShapeDtypeStruct((E, D), jnp.float32)],
        scratch_types=[
            pltpu.VMEM((RW, _IDXW), jnp.int32),
            pltpu.VMEM((GE, D), jnp.float32),
            pltpu.SemaphoreType.DMA,
        ],
    )
    def k(trow_hbm, tcol_hbm, row2d_hbm, col2d_hbm, gr_hbm, gc_hbm,
          idx_v, buf_v, sem):
        c = lax.axis_index("c")
        s = lax.axis_index("s")

        def do_side(t_hbm, i2d_hbm, o_hbm):
            def inner():
                pltpu.sync_copy(i2d_hbm.at[pl.ds(s * RW, RW)], idx_v)

                def grp(g, carry):
                    descs = []
                    for i in range(_GROUP):
                        descs.append(pltpu.async_copy(
                            t_hbm.at[idx_v.at[g * _GROUP + i]],
                            buf_v.at[pl.ds(i * _IDXW, _IDXW)], sem))
                    for d in descs:
                        d.wait()
                    pltpu.sync_copy(buf_v, o_hbm.at[pl.ds(s * EW + g * GE, GE)])
                    return carry

                lax.fori_loop(0, NG, grp, 0)
            return inner

        pl.when(c == 0)(do_side(trow_hbm, row2d_hbm, gr_hbm))
        pl.when(c == 1)(do_side(tcol_hbm, col2d_hbm, gc_hbm))

    return k(table_row, table_col, row2d, col2d)


def _sc_scatter_add(flux, col2d, zeros_nd):
    E, D = flux.shape
    N = zeros_nd.shape[0]
    NW = _NCORE * _NSUB
    EW = E // NW
    RW = EW // _IDXW
    NG = RW // _GROUP
    GE = _GROUP * _IDXW
    ZR = N // _NSUB

    @functools.partial(
        pl.kernel, mesh=_MESH,
        out_type=[jax.ShapeDtypeStruct((N, D), jnp.float32),
                  jax.ShapeDtypeStruct((N, D), jnp.float32)],
        scratch_types=[
            pltpu.VMEM((RW, _IDXW), jnp.int32),
            pltpu.VMEM((GE, D), jnp.float32),
            pltpu.VMEM_SHARED((N, D), jnp.float32),
        ],
    )
    def k(flux_hbm, col2d_hbm, zeros_hbm, out0_hbm, out1_hbm,
          idx_v, fbuf, acc):
        c = lax.axis_index("c")
        s = lax.axis_index("s")
        q = c * _NSUB + s

        pltpu.sync_copy(zeros_hbm.at[pl.ds(s * ZR, ZR)],
                        acc.at[pl.ds(s * ZR, ZR)])
        plsc.subcore_barrier()

        pltpu.sync_copy(col2d_hbm.at[pl.ds(q * RW, RW)], idx_v)

        def grp(g, carry):
            pltpu.sync_copy(flux_hbm.at[pl.ds(q * EW + g * GE, GE)], fbuf)
            for i in range(_GROUP):
                pltpu.sync_copy(fbuf.at[pl.ds(i * _IDXW, _IDXW)],
                                acc.at[idx_v.at[g * _GROUP + i]], add=True)
            return carry

        lax.fori_loop(0, NG, grp, 0)
        plsc.subcore_barrier()

        def write_out(o_hbm):
            def inner():
                pltpu.sync_copy(acc.at[pl.ds(s * ZR, ZR)],
                                o_hbm.at[pl.ds(s * ZR, ZR)])
            return inner

        pl.when(c == 0)(write_out(out0_hbm))
        pl.when(c == 1)(write_out(out1_hbm))

    return k(flux, col2d, zeros_nd)


def kernel(static_nodes, dynamic_nodes, edge_index, edge_features,
           W1, b1, W2, b2, F0, F1, F2):
    N, S = static_nodes.shape
    D = dynamic_nodes.shape[1]
    E = edge_index.shape[1]
    H = W1.shape[1]

    row2d = edge_index[0].reshape(E // _IDXW, _IDXW)
    col2d = edge_index[1].reshape(E // _IDXW, _IDXW)
    zeros_nd = jnp.zeros((N, D), jnp.float32)
    w1e = W1[2 * S + 2 * D:]
    b1r = b1.reshape(1, H)
    b2r = b2.reshape(1, D)

    trow, tcol, h0 = _node_precompute(static_nodes, dynamic_nodes, W1, F0)
    gr, gc = _sc_gather(trow, tcol, row2d, col2d)
    msg, flux0 = _edge_mlp_flux(gr, gc, edge_features, w1e, b1r, W2, b2r)
    p0, p1 = _sc_scatter_add(flux0, col2d, zeros_nd)
    h1 = _update_h(h0, p0, p1, F1)
    hr, hc = _sc_gather(h1, h1, row2d, col2d)
    flux1 = _flux_only(hr, hc, msg)
    p0, p1 = _sc_scatter_add(flux1, col2d, zeros_nd)
    h2 = _update_h(h1, p0, p1, F2)
    return h2


# speedup vs baseline: 8.1065x; 8.1065x over previous
import functools

import jax
import jax.numpy as jnp
from jax import lax
from jax.experimental import pallas as pl
from jax.experimental.pallas import tpu as pltpu
from jax.experimental.pallas import tpu_sc as plsc

_IDXW = 125
_GROUP = 8
_NSUB = 16
_NCORE = 2

_MESH = plsc.VectorSubcoreMesh(
    core_axis_name="c", subcore_axis_name="s",
    num_cores=_NCORE, num_subcores=_NSUB)


def _node_precompute(static_nodes, dynamic_nodes, W1, F0):
    N, S = static_nodes.shape
    D = dynamic_nodes.shape[1]
    H = W1.shape[1]
    BN = 1000

    def body(s_ref, d_ref, w1_ref, f0_ref, trow_ref, tcol_ref, h0_ref):
        s = s_ref[...]
        d = d_ref[...]
        w1 = w1_ref[...]
        f32 = jnp.float32
        prow = (jnp.dot(s, w1[0:S], preferred_element_type=f32)
                + jnp.dot(d, w1[2 * S:2 * S + D], preferred_element_type=f32))
        pcol = (jnp.dot(s, w1[S:2 * S], preferred_element_type=f32)
                + jnp.dot(d, w1[2 * S + D:2 * S + 2 * D], preferred_element_type=f32))
        h0 = jnp.dot(d, f0_ref[...], preferred_element_type=f32)
        trow_ref[...] = prow
        tcol_ref[...] = pcol
        h0_ref[...] = h0

    return pl.pallas_call(
        body,
        grid=(N // BN,),
        in_specs=[
            pl.BlockSpec((BN, S), lambda i: (i, 0)),
            pl.BlockSpec((BN, D), lambda i: (i, 0)),
            pl.BlockSpec(W1.shape, lambda i: (0, 0)),
            pl.BlockSpec((D, D), lambda i: (0, 0)),
        ],
        out_specs=[
            pl.BlockSpec((BN, H), lambda i: (i, 0)),
            pl.BlockSpec((BN, H), lambda i: (i, 0)),
            pl.BlockSpec((BN, D), lambda i: (i, 0)),
        ],
        out_shape=[
            jax.ShapeDtypeStruct((N, H), jnp.float32),
            jax.ShapeDtypeStruct((N, H), jnp.float32),
            jax.ShapeDtypeStruct((N, D), jnp.float32),
        ],
    )(static_nodes, dynamic_nodes, W1, F0)


def _edge_mlp_flux(gr, gc, hr, hc, edge_features, w1e, b1, W2, b2):
    E = gr.shape[0]
    H = W2.shape[0]
    D = W2.shape[1]
    EF = edge_features.shape[1]
    BE = 4000

    def body(gr_ref, gc_ref, hr_ref, hc_ref, ef_ref, w1e_ref, b1_ref,
             w2_ref, b2_ref, msg_ref, flux_ref):
        efb = ef_ref[...]
        pre = gr_ref[...] + gc_ref[...] + b1_ref[...]
        for i in range(EF):
            pre = pre + efb[:, i:i + 1] * w1e_ref[i:i + 1, :]
        m = jnp.dot(jnp.maximum(pre, 0.0), w2_ref[...],
                    preferred_element_type=jnp.float32) + b2_ref[...]
        nrm = jnp.sqrt(jnp.sum(m * m, axis=1, keepdims=True))
        msg = m / nrm
        msg = jnp.where(jnp.isnan(msg), 0.0, msg)
        hr = hr_ref[...]
        hc = hc_ref[...]
        mask = ((jnp.sum(hr, axis=1) != 0.0)
                | (jnp.sum(hc, axis=1) != 0.0))
        flux = jnp.where(mask[:, None], (hc - hr) * msg, 0.0)
        msg_ref[...] = msg
        flux_ref[...] = flux

    return pl.pallas_call(
        body,
        grid=(E // BE,),
        in_specs=[
            pl.BlockSpec((BE, H), lambda i: (i, 0)),
            pl.BlockSpec((BE, H), lambda i: (i, 0)),
            pl.BlockSpec((BE, D), lambda i: (i, 0)),
            pl.BlockSpec((BE, D), lambda i: (i, 0)),
            pl.BlockSpec((BE, EF), lambda i: (i, 0)),
            pl.BlockSpec((EF, H), lambda i: (0, 0)),
            pl.BlockSpec((1, H), lambda i: (0, 0)),
            pl.BlockSpec((H, D), lambda i: (0, 0)),
            pl.BlockSpec((1, D), lambda i: (0, 0)),
        ],
        out_specs=[
            pl.BlockSpec((BE, D), lambda i: (i, 0)),
            pl.BlockSpec((BE, D), lambda i: (i, 0)),
        ],
        out_shape=[
            jax.ShapeDtypeStruct((E, D), jnp.float32),
            jax.ShapeDtypeStruct((E, D), jnp.float32),
        ],
    )(gr, gc, hr, hc, edge_features, w1e, b1, W2, b2)


def _flux_only(hr, hc, msg):
    E, D = hr.shape
    BE = 8000

    def body(hr_ref, hc_ref, msg_ref, flux_ref):
        hrb = hr_ref[...]
        hcb = hc_ref[...]
        mask = ((jnp.sum(hrb, axis=1) != 0.0)
                | (jnp.sum(hcb, axis=1) != 0.0))
        flux_ref[...] = jnp.where(mask[:, None], (hcb - hrb) * msg_ref[...], 0.0)

    return pl.pallas_call(
        body,
        grid=(E // BE,),
        in_specs=[pl.BlockSpec((BE, D), lambda i: (i, 0))] * 3,
        out_specs=pl.BlockSpec((BE, D), lambda i: (i, 0)),
        out_shape=jax.ShapeDtypeStruct((E, D), jnp.float32),
    )(hr, hc, msg)


def _update_h(h, p0, p1, F):
    N, D = h.shape
    BN = 1000

    def body(h_ref, p0_ref, p1_ref, f_ref, out_ref):
        agg = p0_ref[...] + p1_ref[...]
        out_ref[...] = h_ref[...] + jnp.dot(
            agg, f_ref[...], preferred_element_type=jnp.float32)

    return pl.pallas_call(
        body,
        grid=(N // BN,),
        in_specs=[
            pl.BlockSpec((BN, D), lambda i: (i, 0)),
            pl.BlockSpec((BN, D), lambda i: (i, 0)),
            pl.BlockSpec((BN, D), lambda i: (i, 0)),
            pl.BlockSpec((D, D), lambda i: (0, 0)),
        ],
        out_specs=pl.BlockSpec((BN, D), lambda i: (i, 0)),
        out_shape=jax.ShapeDtypeStruct((N, D), jnp.float32),
    )(h, p0, p1, F)


def _sc_gather(tables_row, tables_col, row2d, col2d):
    dims = [t.shape[1] for t in tables_row]
    NT = len(tables_row)
    E = row2d.shape[0] * row2d.shape[1]
    EW = E // _NSUB
    RW = EW // _IDXW
    NG = RW // _GROUP
    GE = _GROUP * _IDXW

    @functools.partial(
        pl.kernel, mesh=_MESH,
        out_type=[jax.ShapeDtypeStruct((E, d), jnp.float32)
                  for d in dims for _ in range(2)],
        scratch_types=[pltpu.VMEM((_GROUP, _IDXW), jnp.int32)]
        + [pltpu.VMEM((GE, d), jnp.float32) for d in dims]
        + [pltpu.SemaphoreType.DMA],
        compiler_params=pltpu.CompilerParams(use_tc_tiling_on_sc=False),
    )
    def k(*refs):
        t_row = refs[0:NT]
        t_col = refs[NT:2 * NT]
        row2d_hbm = refs[2 * NT]
        col2d_hbm = refs[2 * NT + 1]
        outs = refs[2 * NT + 2:2 * NT + 2 + 2 * NT]
        o_row = outs[0::2]
        o_col = outs[1::2]
        idx_v = refs[4 * NT + 2]
        bufs = refs[4 * NT + 3:4 * NT + 3 + NT]
        sem = refs[4 * NT + 3 + NT]

        c = lax.axis_index("c")
        s = lax.axis_index("s")

        def do_side(tabs, i2d_hbm, os_):
            def inner():
                def grp(g, carry):
                    pltpu.sync_copy(
                        i2d_hbm.at[pl.ds(s * RW + g * _GROUP, _GROUP)], idx_v)
                    for t_hbm, buf_v, o_hbm in zip(tabs, bufs, os_):
                        descs = []
                        for i in range(_GROUP):
                            descs.append(pltpu.async_copy(
                                t_hbm.at[idx_v.at[i]],
                                buf_v.at[pl.ds(i * _IDXW, _IDXW)], sem))
                        for d in descs:
                            d.wait()
                        pltpu.sync_copy(
                            buf_v, o_hbm.at[pl.ds(s * EW + g * GE, GE)])
                    return carry

                lax.fori_loop(0, NG, grp, 0)
            return inner

        pl.when(c == 0)(do_side(t_row, row2d_hbm, o_row))
        pl.when(c == 1)(do_side(t_col, col2d_hbm, o_col))

    res = k(*tables_row, *tables_col, row2d, col2d)
    return res


def _sc_scatter_add(flux, col2d, zeros_nd):
    E, D = flux.shape
    NP = zeros_nd.shape[0]
    NW = _NCORE * _NSUB
    SG = 4
    EW = E // NW
    RW = EW // _IDXW
    NG = RW // SG
    GE = SG * _IDXW
    ZR = NP // _NSUB

    @functools.partial(
        pl.kernel, mesh=_MESH,
        out_type=[jax.ShapeDtypeStruct((NP, D), jnp.float32),
                  jax.ShapeDtypeStruct((NP, D), jnp.float32)],
        scratch_types=[
            pltpu.VMEM((SG, _IDXW), jnp.int32),
            pltpu.VMEM((GE, D), jnp.float32),
            pltpu.VMEM_SHARED((NP, D), jnp.float32),
        ],
        compiler_params=pltpu.CompilerParams(use_tc_tiling_on_sc=False),
    )
    def k(flux_hbm, col2d_hbm, zeros_hbm, out0_hbm, out1_hbm,
          idx_v, fbuf, acc):
        c = lax.axis_index("c")
        s = lax.axis_index("s")
        q = c * _NSUB + s

        pltpu.sync_copy(zeros_hbm.at[pl.ds(s * ZR, ZR)],
                        acc.at[pl.ds(s * ZR, ZR)])
        plsc.subcore_barrier()

        def grp(g, carry):
            pltpu.sync_copy(col2d_hbm.at[pl.ds(q * RW + g * SG, SG)], idx_v)
            pltpu.sync_copy(flux_hbm.at[pl.ds(q * EW + g * GE, GE)], fbuf)
            for i in range(SG):
                pltpu.sync_copy(fbuf.at[pl.ds(i * _IDXW, _IDXW)],
                                acc.at[idx_v.at[i]], add=True)
            return carry

        lax.fori_loop(0, NG, grp, 0)
        plsc.subcore_barrier()

        def write_out(o_hbm):
            def inner():
                pltpu.sync_copy(acc.at[pl.ds(s * ZR, ZR)],
                                o_hbm.at[pl.ds(s * ZR, ZR)])
            return inner

        pl.when(c == 0)(write_out(out0_hbm))
        pl.when(c == 1)(write_out(out1_hbm))

    return k(flux, col2d, zeros_nd)


def kernel(static_nodes, dynamic_nodes, edge_index, edge_features,
           W1, b1, W2, b2, F0, F1, F2):
    N, S = static_nodes.shape
    D = dynamic_nodes.shape[1]
    E = edge_index.shape[1]
    H = W1.shape[1]

    row2d = edge_index[0].reshape(E // _IDXW, _IDXW)
    col2d = edge_index[1].reshape(E // _IDXW, _IDXW)
    npad = 8 * _NSUB
    NP = ((N + npad - 1) // npad) * npad
    zeros_nd = jnp.zeros((NP, D), jnp.float32)
    w1e = W1[2 * S + 2 * D:]
    b1r = b1.reshape(1, H)
    b2r = b2.reshape(1, D)

    trow, tcol, h0 = _node_precompute(static_nodes, dynamic_nodes, W1, F0)
    gr, gc, hr0, hc0 = _sc_gather((trow, h0), (tcol, h0), row2d, col2d)
    msg, flux0 = _edge_mlp_flux(gr, gc, hr0, hc0, edge_features,
                                w1e, b1r, W2, b2r)
    p0, p1 = _sc_scatter_add(flux0, col2d, zeros_nd)
    h1 = _update_h(h0, p0, p1, F1)
    hr, hc = _sc_gather((h1,), (h1,), row2d, col2d)
    flux1 = _flux_only(hr, hc, msg)
    p0, p1 = _sc_scatter_add(flux1, col2d, zeros_nd)
    h2 = _update_h(h1, p0, p1, F2)
    return h2


# speedup vs baseline: 8.3893x; 1.0349x over previous
import functools

import jax
import jax.numpy as jnp
from jax import lax
from jax.experimental import pallas as pl
from jax.experimental.pallas import tpu as pltpu
from jax.experimental.pallas import tpu_sc as plsc

_IDXW = 125
_GROUP = 8
_NSUB = 16
_NCORE = 2

_MESH = plsc.VectorSubcoreMesh(
    core_axis_name="c", subcore_axis_name="s",
    num_cores=_NCORE, num_subcores=_NSUB)


def _node_precompute(static_nodes, dynamic_nodes, W1, F0):
    N, S = static_nodes.shape
    D = dynamic_nodes.shape[1]
    H = W1.shape[1]
    BN = 1000

    def body(s_ref, d_ref, w1_ref, f0_ref, trow_ref, tcol_ref, h0_ref):
        s = s_ref[...]
        d = d_ref[...]
        w1 = w1_ref[...]
        f32 = jnp.float32
        prow = (jnp.dot(s, w1[0:S], preferred_element_type=f32)
                + jnp.dot(d, w1[2 * S:2 * S + D], preferred_element_type=f32))
        pcol = (jnp.dot(s, w1[S:2 * S], preferred_element_type=f32)
                + jnp.dot(d, w1[2 * S + D:2 * S + 2 * D], preferred_element_type=f32))
        h0 = jnp.dot(d, f0_ref[...], preferred_element_type=f32)
        trow_ref[...] = prow
        tcol_ref[...] = pcol
        h0_ref[...] = h0

    return pl.pallas_call(
        body,
        grid=(N // BN,),
        in_specs=[
            pl.BlockSpec((BN, S), lambda i: (i, 0)),
            pl.BlockSpec((BN, D), lambda i: (i, 0)),
            pl.BlockSpec(W1.shape, lambda i: (0, 0)),
            pl.BlockSpec((D, D), lambda i: (0, 0)),
        ],
        out_specs=[
            pl.BlockSpec((BN, H), lambda i: (i, 0)),
            pl.BlockSpec((BN, H), lambda i: (i, 0)),
            pl.BlockSpec((BN, D), lambda i: (i, 0)),
        ],
        out_shape=[
            jax.ShapeDtypeStruct((N, H), jnp.float32),
            jax.ShapeDtypeStruct((N, H), jnp.float32),
            jax.ShapeDtypeStruct((N, D), jnp.float32),
        ],
    )(static_nodes, dynamic_nodes, W1, F0)


def _edge_mlp_flux(gr, gc, hr, hc, edge_features, w1e, b1, W2, b2):
    E = gr.shape[0]
    H = W2.shape[0]
    D = W2.shape[1]
    EF = edge_features.shape[1]
    BE = 4000

    def body(gr_ref, gc_ref, hr_ref, hc_ref, ef_ref, w1e_ref, b1_ref,
             w2_ref, b2_ref, msg_ref, flux_ref):
        efb = ef_ref[...]
        pre = gr_ref[...] + gc_ref[...] + b1_ref[...]
        for i in range(EF):
            pre = pre + efb[:, i:i + 1] * w1e_ref[i:i + 1, :]
        m = jnp.dot(jnp.maximum(pre, 0.0), w2_ref[...],
                    preferred_element_type=jnp.float32) + b2_ref[...]
        nrm = jnp.sqrt(jnp.sum(m * m, axis=1, keepdims=True))
        msg = m / nrm
        msg = jnp.where(jnp.isnan(msg), 0.0, msg)
        hr = hr_ref[...]
        hc = hc_ref[...]
        mask = ((jnp.sum(hr, axis=1) != 0.0)
                | (jnp.sum(hc, axis=1) != 0.0))
        flux = jnp.where(mask[:, None], (hc - hr) * msg, 0.0)
        msg_ref[...] = msg
        flux_ref[...] = flux

    return pl.pallas_call(
        body,
        grid=(E // BE,),
        in_specs=[
            pl.BlockSpec((BE, H), lambda i: (i, 0)),
            pl.BlockSpec((BE, H), lambda i: (i, 0)),
            pl.BlockSpec((BE, D), lambda i: (i, 0)),
            pl.BlockSpec((BE, D), lambda i: (i, 0)),
            pl.BlockSpec((BE, EF), lambda i: (i, 0)),
            pl.BlockSpec((EF, H), lambda i: (0, 0)),
            pl.BlockSpec((1, H), lambda i: (0, 0)),
            pl.BlockSpec((H, D), lambda i: (0, 0)),
            pl.BlockSpec((1, D), lambda i: (0, 0)),
        ],
        out_specs=[
            pl.BlockSpec((BE, D), lambda i: (i, 0)),
            pl.BlockSpec((BE, D), lambda i: (i, 0)),
        ],
        out_shape=[
            jax.ShapeDtypeStruct((E, D), jnp.float32),
            jax.ShapeDtypeStruct((E, D), jnp.float32),
        ],
    )(gr, gc, hr, hc, edge_features, w1e, b1, W2, b2)


def _flux_only(hr, hc, msg):
    E, D = hr.shape
    BE = 8000

    def body(hr_ref, hc_ref, msg_ref, flux_ref):
        hrb = hr_ref[...]
        hcb = hc_ref[...]
        mask = ((jnp.sum(hrb, axis=1) != 0.0)
                | (jnp.sum(hcb, axis=1) != 0.0))
        flux_ref[...] = jnp.where(mask[:, None], (hcb - hrb) * msg_ref[...], 0.0)

    return pl.pallas_call(
        body,
        grid=(E // BE,),
        in_specs=[pl.BlockSpec((BE, D), lambda i: (i, 0))] * 3,
        out_specs=pl.BlockSpec((BE, D), lambda i: (i, 0)),
        out_shape=jax.ShapeDtypeStruct((E, D), jnp.float32),
    )(hr, hc, msg)


def _update_h(h, p0, p1, F):
    N, D = h.shape
    BN = 1000

    def body(h_ref, p0_ref, p1_ref, f_ref, out_ref):
        agg = p0_ref[...] + p1_ref[...]
        out_ref[...] = h_ref[...] + jnp.dot(
            agg, f_ref[...], preferred_element_type=jnp.float32)

    return pl.pallas_call(
        body,
        grid=(N // BN,),
        in_specs=[
            pl.BlockSpec((BN, D), lambda i: (i, 0)),
            pl.BlockSpec((BN, D), lambda i: (i, 0)),
            pl.BlockSpec((BN, D), lambda i: (i, 0)),
            pl.BlockSpec((D, D), lambda i: (0, 0)),
        ],
        out_specs=pl.BlockSpec((BN, D), lambda i: (i, 0)),
        out_shape=jax.ShapeDtypeStruct((N, D), jnp.float32),
    )(h, p0, p1, F)


def _sc_gather(tables_row, tables_col, row2d, col2d, group):
    dims = [t.shape[1] for t in tables_row]
    NT = len(tables_row)
    E = row2d.shape[0] * row2d.shape[1]
    EW = E // _NSUB
    RW = EW // _IDXW
    NG = RW // group
    GE = group * _IDXW

    @functools.partial(
        pl.kernel, mesh=_MESH,
        out_type=[jax.ShapeDtypeStruct((E, d), jnp.float32)
                  for d in dims for _ in range(2)],
        scratch_types=[pltpu.VMEM((2, group, _IDXW), jnp.int32)]
        + [pltpu.VMEM((2, GE, d), jnp.float32) for d in dims]
        + [pltpu.SemaphoreType.DMA],
        compiler_params=pltpu.CompilerParams(use_tc_tiling_on_sc=False),
    )
    def k(*refs):
        t_row = refs[0:NT]
        t_col = refs[NT:2 * NT]
        row2d_hbm = refs[2 * NT]
        col2d_hbm = refs[2 * NT + 1]
        outs = refs[2 * NT + 2:2 * NT + 2 + 2 * NT]
        o_row = outs[0::2]
        o_col = outs[1::2]
        idx_v = refs[4 * NT + 2]
        bufs = refs[4 * NT + 3:4 * NT + 3 + NT]
        sem = refs[4 * NT + 3 + NT]

        c = lax.axis_index("c")
        s = lax.axis_index("s")

        def do_side(tabs, i2d_hbm, os_):
            def inner():
                def load_idx(g, b):
                    pltpu.sync_copy(
                        i2d_hbm.at[pl.ds(s * RW + g * group, group)],
                        idx_v.at[b])

                def fire(b):
                    for t_hbm, buf_v in zip(tabs, bufs):
                        for i in range(group):
                            pltpu.async_copy(
                                t_hbm.at[idx_v.at[b].at[i]],
                                buf_v.at[b].at[pl.ds(i * _IDXW, _IDXW)], sem)

                def drain(b):
                    for t_hbm, buf_v in zip(tabs, bufs):
                        pltpu.make_async_copy(
                            t_hbm.at[pl.ds(0, GE)], buf_v.at[b], sem).wait()

                def write_out(g, b):
                    for buf_v, o_hbm in zip(bufs, os_):
                        pltpu.sync_copy(
                            buf_v.at[b], o_hbm.at[pl.ds(s * EW + g * GE, GE)])

                load_idx(0, 0)
                fire(0)

                def pair(p, carry):
                    for b in (0, 1):
                        g = p * 2 + b

                        @pl.when(g + 1 < NG)
                        def _():
                            load_idx(g + 1, 1 - b)
                            fire(1 - b)

                        drain(b)
                        write_out(g, b)
                    return carry

                lax.fori_loop(0, NG // 2, pair, 0)
            return inner

        pl.when(c == 0)(do_side(t_row, row2d_hbm, o_row))
        pl.when(c == 1)(do_side(t_col, col2d_hbm, o_col))

    res = k(*tables_row, *tables_col, row2d, col2d)
    return res


def _sc_scatter_add(flux, col2d, zeros_nd):
    E, D = flux.shape
    NP = zeros_nd.shape[0]
    NW = _NCORE * _NSUB
    SG = 2
    EW = E // NW
    RW = EW // _IDXW
    NG = RW // SG
    GE = SG * _IDXW
    ZR = NP // _NSUB

    @functools.partial(
        pl.kernel, mesh=_MESH,
        out_type=[jax.ShapeDtypeStruct((NP, D), jnp.float32),
                  jax.ShapeDtypeStruct((NP, D), jnp.float32)],
        scratch_types=[
            pltpu.VMEM((2, SG, _IDXW), jnp.int32),
            pltpu.VMEM((2, GE, D), jnp.float32),
            pltpu.VMEM_SHARED((NP, D), jnp.float32),
            pltpu.SemaphoreType.DMA,
            pltpu.SemaphoreType.DMA,
        ],
        compiler_params=pltpu.CompilerParams(use_tc_tiling_on_sc=False),
    )
    def k(flux_hbm, col2d_hbm, zeros_hbm, out0_hbm, out1_hbm,
          idx_v, fbuf, acc, sem_f, sem_s):
        c = lax.axis_index("c")
        s = lax.axis_index("s")
        q = c * _NSUB + s

        pltpu.sync_copy(zeros_hbm.at[pl.ds(s * ZR, ZR)],
                        acc.at[pl.ds(s * ZR, ZR)])
        plsc.subcore_barrier()

        def load_idx(g, b):
            pltpu.sync_copy(col2d_hbm.at[pl.ds(q * RW + g * SG, SG)],
                            idx_v.at[b])

        def load_flux(g, b):
            pltpu.async_copy(flux_hbm.at[pl.ds(q * EW + g * GE, GE)],
                             fbuf.at[b], sem_f)

        def drain(b, sem):
            pltpu.make_async_copy(flux_hbm.at[pl.ds(0, GE)],
                                  fbuf.at[b], sem).wait()

        def fire_scatter(b):
            for i in range(SG):
                pltpu.async_copy(fbuf.at[b].at[pl.ds(i * _IDXW, _IDXW)],
                                 acc.at[idx_v.at[b].at[i]], sem_s, add=True)

        load_idx(0, 0)
        load_flux(0, 0)

        def pair(p, carry):
            for b in (0, 1):
                g = p * 2 + b

                @pl.when(g + 1 < NG)
                def _():
                    load_idx(g + 1, 1 - b)

                @pl.when(g > 0)
                def _():
                    drain(1 - b, sem_s)

                @pl.when(g + 1 < NG)
                def _():
                    load_flux(g + 1, 1 - b)

                drain(b, sem_f)
                fire_scatter(b)
            return carry

        lax.fori_loop(0, NG // 2, pair, 0)
        drain((NG - 1) % 2, sem_s)
        plsc.subcore_barrier()

        def write_out(o_hbm):
            def inner():
                pltpu.sync_copy(acc.at[pl.ds(s * ZR, ZR)],
                                o_hbm.at[pl.ds(s * ZR, ZR)])
            return inner

        pl.when(c == 0)(write_out(out0_hbm))
        pl.when(c == 1)(write_out(out1_hbm))

    return k(flux, col2d, zeros_nd)


def kernel(static_nodes, dynamic_nodes, edge_index, edge_features,
           W1, b1, W2, b2, F0, F1, F2):
    N, S = static_nodes.shape
    D = dynamic_nodes.shape[1]
    E = edge_index.shape[1]
    H = W1.shape[1]

    row2d = edge_index[0].reshape(E // _IDXW, _IDXW)
    col2d = edge_index[1].reshape(E // _IDXW, _IDXW)
    npad = 8 * _NSUB
    NP = ((N + npad - 1) // npad) * npad
    zeros_nd = jnp.zeros((NP, D), jnp.float32)
    w1e = W1[2 * S + 2 * D:]
    b1r = b1.reshape(1, H)
    b2r = b2.reshape(1, D)

    trow, tcol, h0 = _node_precompute(static_nodes, dynamic_nodes, W1, F0)
    gr, gc, hr0, hc0 = _sc_gather((trow, h0), (tcol, h0), row2d, col2d,
                                  group=4)
    msg, flux0 = _edge_mlp_flux(gr, gc, hr0, hc0, edge_features,
                                w1e, b1r, W2, b2r)
    p0, p1 = _sc_scatter_add(flux0, col2d, zeros_nd)
    h1 = _update_h(h0, p0, p1, F1)
    hr, hc = _sc_gather((h1,), (h1,), row2d, col2d, group=8)
    flux1 = _flux_only(hr, hc, msg)
    p0, p1 = _sc_scatter_add(flux1, col2d, zeros_nd)
    h2 = _update_h(h1, p0, p1, F2)
    return h2


# speedup vs baseline: 8.8049x; 1.0495x over previous
import functools

import jax
import jax.numpy as jnp
from jax import lax
from jax.experimental import pallas as pl
from jax.experimental.pallas import tpu as pltpu
from jax.experimental.pallas import tpu_sc as plsc

_IDXW = 125
_GROUP = 8
_NSUB = 16
_NCORE = 2

_MESH = plsc.VectorSubcoreMesh(
    core_axis_name="c", subcore_axis_name="s",
    num_cores=_NCORE, num_subcores=_NSUB)


def _edge_mlp_flux(sr, sc_, dr, dc, edge_features, W1, b1, W2, b2, F0, F1):
    E = sr.shape[0]
    S = sr.shape[1]
    D = dr.shape[1]
    H = W2.shape[0]
    EF = edge_features.shape[1]
    BE = 4000

    def body(sr_ref, sc_ref, dr_ref, dc_ref, ef_ref, w1_ref, b1_ref,
             w2_ref, b2_ref, f0_ref, f1_ref, msg_ref, flux_ref):
        f32 = jnp.float32
        w1 = w1_ref[...]
        efb = ef_ref[...]
        drb = dr_ref[...]
        dcb = dc_ref[...]
        pre = (jnp.dot(sr_ref[...], w1[0:S], preferred_element_type=f32)
               + jnp.dot(sc_ref[...], w1[S:2 * S], preferred_element_type=f32)
               + jnp.dot(drb, w1[2 * S:2 * S + D], preferred_element_type=f32)
               + jnp.dot(dcb, w1[2 * S + D:2 * S + 2 * D],
                         preferred_element_type=f32)
               + b1_ref[...])
        for i in range(EF):
            pre = pre + efb[:, i:i + 1] * w1[2 * S + 2 * D + i:2 * S + 2 * D + i + 1, :]
        m = jnp.dot(jnp.maximum(pre, 0.0), w2_ref[...],
                    preferred_element_type=f32) + b2_ref[...]
        nrm = jnp.sqrt(jnp.sum(m * m, axis=1, keepdims=True))
        msg = m / nrm
        msg = jnp.where(jnp.isnan(msg), 0.0, msg)
        hr = jnp.dot(drb, f0_ref[...], preferred_element_type=f32)
        hc = jnp.dot(dcb, f0_ref[...], preferred_element_type=f32)
        mask = ((jnp.sum(hr, axis=1) != 0.0)
                | (jnp.sum(hc, axis=1) != 0.0))
        flux = jnp.where(mask[:, None], (hc - hr) * msg, 0.0)
        msg_ref[...] = msg
        flux_ref[...] = jnp.dot(flux, f1_ref[...], preferred_element_type=f32)

    return pl.pallas_call(
        body,
        grid=(E // BE,),
        in_specs=[
            pl.BlockSpec((BE, S), lambda i: (i, 0)),
            pl.BlockSpec((BE, S), lambda i: (i, 0)),
            pl.BlockSpec((BE, D), lambda i: (i, 0)),
            pl.BlockSpec((BE, D), lambda i: (i, 0)),
            pl.BlockSpec((BE, EF), lambda i: (i, 0)),
            pl.BlockSpec(W1.shape, lambda i: (0, 0)),
            pl.BlockSpec((1, H), lambda i: (0, 0)),
            pl.BlockSpec((H, D), lambda i: (0, 0)),
            pl.BlockSpec((1, D), lambda i: (0, 0)),
            pl.BlockSpec((D, D), lambda i: (0, 0)),
            pl.BlockSpec((D, D), lambda i: (0, 0)),
        ],
        out_specs=[
            pl.BlockSpec((BE, D), lambda i: (i, 0)),
            pl.BlockSpec((BE, D), lambda i: (i, 0)),
        ],
        out_shape=[
            jax.ShapeDtypeStruct((E, D), jnp.float32),
            jax.ShapeDtypeStruct((E, D), jnp.float32),
        ],
    )(sr, sc_, dr, dc, edge_features, W1, b1, W2, b2, F0, F1)


def _flux_only(hr, hc, msg, F):
    E, D = hr.shape
    BE = 8000

    def body(hr_ref, hc_ref, msg_ref, f_ref, flux_ref):
        hrb = hr_ref[...]
        hcb = hc_ref[...]
        mask = ((jnp.sum(hrb, axis=1) != 0.0)
                | (jnp.sum(hcb, axis=1) != 0.0))
        flux = jnp.where(mask[:, None], (hcb - hrb) * msg_ref[...], 0.0)
        flux_ref[...] = jnp.dot(flux, f_ref[...],
                                preferred_element_type=jnp.float32)

    return pl.pallas_call(
        body,
        grid=(E // BE,),
        in_specs=[pl.BlockSpec((BE, D), lambda i: (i, 0))] * 3
        + [pl.BlockSpec((D, D), lambda i: (0, 0))],
        out_specs=pl.BlockSpec((BE, D), lambda i: (i, 0)),
        out_shape=jax.ShapeDtypeStruct((E, D), jnp.float32),
    )(hr, hc, msg, F)


def _update_h1(dyn, p0, p1, F0):
    N, D = dyn.shape
    BN = 1000

    def body(d_ref, p0_ref, p1_ref, f0_ref, out_ref):
        out_ref[...] = (jnp.dot(d_ref[...], f0_ref[...],
                                preferred_element_type=jnp.float32)
                        + p0_ref[...] + p1_ref[...])

    return pl.pallas_call(
        body,
        grid=(N // BN,),
        in_specs=[
            pl.BlockSpec((BN, D), lambda i: (i, 0)),
            pl.BlockSpec((BN, D), lambda i: (i, 0)),
            pl.BlockSpec((BN, D), lambda i: (i, 0)),
            pl.BlockSpec((D, D), lambda i: (0, 0)),
        ],
        out_specs=pl.BlockSpec((BN, D), lambda i: (i, 0)),
        out_shape=jax.ShapeDtypeStruct((N, D), jnp.float32),
    )(dyn, p0, p1, F0)


def _update_h2(h, q0, q1):
    N, D = h.shape
    BN = 1000

    def body(h_ref, q0_ref, q1_ref, out_ref):
        out_ref[...] = h_ref[...] + q0_ref[...] + q1_ref[...]

    return pl.pallas_call(
        body,
        grid=(N // BN,),
        in_specs=[
            pl.BlockSpec((BN, D), lambda i: (i, 0)),
            pl.BlockSpec((BN, D), lambda i: (i, 0)),
            pl.BlockSpec((BN, D), lambda i: (i, 0)),
        ],
        out_specs=pl.BlockSpec((BN, D), lambda i: (i, 0)),
        out_shape=jax.ShapeDtypeStruct((N, D), jnp.float32),
    )(h, q0, q1)


def _sc_gather(tables_row, tables_col, row2d, col2d, group):
    dims = [t.shape[1] for t in tables_row]
    NT = len(tables_row)
    E = row2d.shape[0] * row2d.shape[1]
    EW = E // _NSUB
    RW = EW // _IDXW
    NG = RW // group
    GE = group * _IDXW

    @functools.partial(
        pl.kernel, mesh=_MESH,
        out_type=[jax.ShapeDtypeStruct((E, d), jnp.float32)
                  for d in dims for _ in range(2)],
        scratch_types=[pltpu.VMEM((2, group, _IDXW), jnp.int32)]
        + [pltpu.VMEM((2, GE, d), jnp.float32) for d in dims]
        + [pltpu.SemaphoreType.DMA],
        compiler_params=pltpu.CompilerParams(use_tc_tiling_on_sc=False),
    )
    def k(*refs):
        t_row = refs[0:NT]
        t_col = refs[NT:2 * NT]
        row2d_hbm = refs[2 * NT]
        col2d_hbm = refs[2 * NT + 1]
        outs = refs[2 * NT + 2:2 * NT + 2 + 2 * NT]
        o_row = outs[0::2]
        o_col = outs[1::2]
        idx_v = refs[4 * NT + 2]
        bufs = refs[4 * NT + 3:4 * NT + 3 + NT]
        sem = refs[4 * NT + 3 + NT]

        c = lax.axis_index("c")
        s = lax.axis_index("s")

        def do_side(tabs, i2d_hbm, os_):
            def inner():
                def load_idx(g, b):
                    pltpu.sync_copy(
                        i2d_hbm.at[pl.ds(s * RW + g * group, group)],
                        idx_v.at[b])

                def fire(b):
                    for t_hbm, buf_v in zip(tabs, bufs):
                        for i in range(group):
                            pltpu.async_copy(
                                t_hbm.at[idx_v.at[b].at[i]],
                                buf_v.at[b].at[pl.ds(i * _IDXW, _IDXW)], sem)

                def drain(b):
                    for t_hbm, buf_v in zip(tabs, bufs):
                        pltpu.make_async_copy(
                            t_hbm.at[pl.ds(0, GE)], buf_v.at[b], sem).wait()

                def write_out(g, b):
                    for buf_v, o_hbm in zip(bufs, os_):
                        pltpu.sync_copy(
                            buf_v.at[b], o_hbm.at[pl.ds(s * EW + g * GE, GE)])

                load_idx(0, 0)
                fire(0)

                def pair(p, carry):
                    for b in (0, 1):
                        g = p * 2 + b

                        @pl.when(g + 1 < NG)
                        def _():
                            load_idx(g + 1, 1 - b)
                            fire(1 - b)

                        drain(b)
                        write_out(g, b)
                    return carry

                lax.fori_loop(0, NG // 2, pair, 0)
            return inner

        pl.when(c == 0)(do_side(t_row, row2d_hbm, o_row))
        pl.when(c == 1)(do_side(t_col, col2d_hbm, o_col))

    res = k(*tables_row, *tables_col, row2d, col2d)
    return res


def _sc_scatter_add(flux, col2d, zeros_nd):
    E, D = flux.shape
    NP = zeros_nd.shape[0]
    NW = _NCORE * _NSUB
    SG = 2
    EW = E // NW
    RW = EW // _IDXW
    NG = RW // SG
    GE = SG * _IDXW
    ZR = NP // _NSUB

    @functools.partial(
        pl.kernel, mesh=_MESH,
        out_type=[jax.ShapeDtypeStruct((NP, D), jnp.float32),
                  jax.ShapeDtypeStruct((NP, D), jnp.float32)],
        scratch_types=[
            pltpu.VMEM((2, SG, _IDXW), jnp.int32),
            pltpu.VMEM((2, GE, D), jnp.float32),
            pltpu.VMEM_SHARED((NP, D), jnp.float32),
            pltpu.SemaphoreType.DMA,
            pltpu.SemaphoreType.DMA,
        ],
        compiler_params=pltpu.CompilerParams(use_tc_tiling_on_sc=False),
    )
    def k(flux_hbm, col2d_hbm, zeros_hbm, out0_hbm, out1_hbm,
          idx_v, fbuf, acc, sem_f, sem_s):
        c = lax.axis_index("c")
        s = lax.axis_index("s")
        q = c * _NSUB + s

        pltpu.sync_copy(zeros_hbm.at[pl.ds(s * ZR, ZR)],
                        acc.at[pl.ds(s * ZR, ZR)])
        plsc.subcore_barrier()

        def load_idx(g, b):
            pltpu.sync_copy(col2d_hbm.at[pl.ds(q * RW + g * SG, SG)],
                            idx_v.at[b])

        def load_flux(g, b):
            pltpu.async_copy(flux_hbm.at[pl.ds(q * EW + g * GE, GE)],
                             fbuf.at[b], sem_f)

        def drain(b, sem):
            pltpu.make_async_copy(flux_hbm.at[pl.ds(0, GE)],
                                  fbuf.at[b], sem).wait()

        def fire_scatter(b):
            for i in range(SG):
                pltpu.async_copy(fbuf.at[b].at[pl.ds(i * _IDXW, _IDXW)],
                                 acc.at[idx_v.at[b].at[i]], sem_s, add=True)

        load_idx(0, 0)
        load_flux(0, 0)

        def pair(p, carry):
            for b in (0, 1):
                g = p * 2 + b

                @pl.when(g + 1 < NG)
                def _():
                    load_idx(g + 1, 1 - b)

                @pl.when(g > 0)
                def _():
                    drain(1 - b, sem_s)

                @pl.when(g + 1 < NG)
                def _():
                    load_flux(g + 1, 1 - b)

                drain(b, sem_f)
                fire_scatter(b)
            return carry

        lax.fori_loop(0, NG // 2, pair, 0)
        drain((NG - 1) % 2, sem_s)
        plsc.subcore_barrier()

        def write_out(o_hbm):
            def inner():
                pltpu.sync_copy(acc.at[pl.ds(s * ZR, ZR)],
                                o_hbm.at[pl.ds(s * ZR, ZR)])
            return inner

        pl.when(c == 0)(write_out(out0_hbm))
        pl.when(c == 1)(write_out(out1_hbm))

    return k(flux, col2d, zeros_nd)


def kernel(static_nodes, dynamic_nodes, edge_index, edge_features,
           W1, b1, W2, b2, F0, F1, F2):
    N, S = static_nodes.shape
    D = dynamic_nodes.shape[1]
    E = edge_index.shape[1]
    H = W1.shape[1]

    row2d = edge_index[0].reshape(E // _IDXW, _IDXW)
    col2d = edge_index[1].reshape(E // _IDXW, _IDXW)
    npad = 8 * _NSUB
    NP = ((N + npad - 1) // npad) * npad
    zeros_nd = jnp.zeros((NP, D), jnp.float32)
    b1r = b1.reshape(1, H)
    b2r = b2.reshape(1, D)

    sr, sc_, dr, dc = _sc_gather((static_nodes, dynamic_nodes),
                                 (static_nodes, dynamic_nodes),
                                 row2d, col2d, group=4)
    msg, flux0 = _edge_mlp_flux(sr, sc_, dr, dc, edge_features,
                                W1, b1r, W2, b2r, F0, F1)
    p0, p1 = _sc_scatter_add(flux0, col2d, zeros_nd)
    h1 = _update_h1(dynamic_nodes, p0, p1, F0)
    hr, hc = _sc_gather((h1,), (h1,), row2d, col2d, group=8)
    flux1 = _flux_only(hr, hc, msg, F2)
    q0, q1 = _sc_scatter_add(flux1, col2d, zeros_nd)
    h2 = _update_h2(h1, q0, q1)
    return h2
